# baseline (speedup 1.0000x reference)
import jax
import jax.numpy as jnp
from jax.experimental import pallas as pl
from jax.experimental.pallas import tpu as pltpu
from jax.experimental.pallas import tpu_sc as plsc


def _sc_gather_pairs(table2, ids2):
    batch = ids2.shape[0]
    row = table2.shape[1]
    window = 128
    indices = ids2.reshape(1, batch)
    mesh = plsc.VectorSubcoreMesh(core_axis_name="core", subcore_axis_name="subcore")

    @pl.kernel(
        out_type=jax.ShapeDtypeStruct((batch, row), table2.dtype),
        mesh=mesh,
    )
    def gather_kernel(tbl_hbm, idx_hbm, out_hbm):
        def body(idx_vmem, out_vmem):
            pltpu.sync_copy(tbl_hbm.at[idx_vmem.at[0]], out_vmem)

        pltpu.emit_pipeline(
            body,
            grid=(batch // window,),
            in_specs=[pl.BlockSpec((1, window), index_map=lambda i: (0, i))],
            out_specs=[pl.BlockSpec((window, row), index_map=lambda i: (i, 0))],
            core_axis_name="subcore",
            dimension_semantics=(pltpu.PARALLEL,),
        )(idx_hbm, out_hbm)

    return gather_kernel(table2, indices)


def _tc_head(x_pair, parity, head_w):
    b = x_pair.shape[0]
    v, d = head_w.shape
    vt = 2048

    def mm_kernel(xp_ref, par_ref, w_ref, o_ref):
        par = par_ref[...]
        x = xp_ref[:, :d] * (1.0 - par) + xp_ref[:, d:] * par
        o_ref[...] = jax.lax.dot_general(
            x,
            w_ref[...],
            dimension_numbers=(((1,), (1,)), ((), ())),
            preferred_element_type=jnp.float32,
            precision=jax.lax.Precision.HIGHEST,
        )

    return pl.pallas_call(
        mm_kernel,
        grid=(pl.cdiv(v, vt),),
        in_specs=[
            pl.BlockSpec((b, 2 * d), lambda i: (0, 0)),
            pl.BlockSpec((b, 1), lambda i: (0, 0)),
            pl.BlockSpec((vt, d), lambda i: (i, 0)),
        ],
        out_specs=pl.BlockSpec((b, vt), lambda i: (0, i)),
        out_shape=jax.ShapeDtypeStruct((b, v), jnp.float32),
        compiler_params=pltpu.CompilerParams(dimension_semantics=("parallel",)),
    )(x_pair, parity, head_w)


def kernel(input_ids, embed_table, head_w):
    v, d = embed_table.shape
    table2 = embed_table.reshape(v // 2, 2 * d)
    ids2 = jax.lax.shift_right_logical(input_ids, 1)
    parity = (input_ids & 1).astype(jnp.float32).reshape(-1, 1)
    x_pair = _sc_gather_pairs(table2, ids2)
    return _tc_head(x_pair, parity, head_w)


# speedup vs baseline: 1.3432x; 1.3432x over previous
import jax
import jax.numpy as jnp
from jax.experimental import pallas as pl
from jax.experimental.pallas import tpu as pltpu
from jax.experimental.pallas import tpu_sc as plsc


def _sc_gather_pairs(table2, ids2):
    batch = ids2.shape[0]
    row = table2.shape[1]
    window = 128
    indices = ids2.reshape(1, batch)
    mesh = plsc.VectorSubcoreMesh(core_axis_name="core", subcore_axis_name="subcore")

    @pl.kernel(
        out_type=jax.ShapeDtypeStruct((batch, row), table2.dtype),
        mesh=mesh,
    )
    def gather_kernel(tbl_hbm, idx_hbm, out_hbm):
        def body(idx_vmem, out_vmem):
            pltpu.sync_copy(tbl_hbm.at[idx_vmem.at[0]], out_vmem)

        pltpu.emit_pipeline(
            body,
            grid=(batch // window,),
            in_specs=[pl.BlockSpec((1, window), index_map=lambda i: (0, i))],
            out_specs=[pl.BlockSpec((window, row), index_map=lambda i: (i, 0))],
            core_axis_name="subcore",
            dimension_semantics=(pltpu.PARALLEL,),
        )(idx_hbm, out_hbm)

    return gather_kernel(table2, indices)


def _tc_head(x_pair, parity, head_w):
    b = x_pair.shape[0]
    v, d = head_w.shape
    vt = 4096

    def mm_kernel(xp_ref, par_ref, w_ref, o_ref):
        par = par_ref[...]
        x = (xp_ref[:, :d] * (1.0 - par) + xp_ref[:, d:] * par).astype(jnp.bfloat16)
        o_ref[...] = jax.lax.dot_general(
            x,
            w_ref[...].astype(jnp.bfloat16),
            dimension_numbers=(((1,), (1,)), ((), ())),
            preferred_element_type=jnp.float32,
        )

    return pl.pallas_call(
        mm_kernel,
        grid=(pl.cdiv(v, vt),),
        in_specs=[
            pl.BlockSpec((b, 2 * d), lambda i: (0, 0)),
            pl.BlockSpec((b, 1), lambda i: (0, 0)),
            pl.BlockSpec((vt, d), lambda i: (i, 0)),
        ],
        out_specs=pl.BlockSpec((b, vt), lambda i: (0, i)),
        out_shape=jax.ShapeDtypeStruct((b, v), jnp.float32),
        compiler_params=pltpu.CompilerParams(dimension_semantics=("parallel",)),
    )(x_pair, parity, head_w)


def kernel(input_ids, embed_table, head_w):
    v, d = embed_table.shape
    table2 = embed_table.reshape(v // 2, 2 * d)
    ids2 = jax.lax.shift_right_logical(input_ids, 1)
    parity = (input_ids & 1).astype(jnp.float32).reshape(-1, 1)
    x_pair = _sc_gather_pairs(table2, ids2)
    return _tc_head(x_pair, parity, head_w)
